# baseline (speedup 1.0000x reference)
import functools

import jax
import jax.numpy as jnp
from jax import lax
from jax.experimental import pallas as pl
from jax.experimental.pallas import tpu as pltpu
from jax.experimental.pallas import tpu_sc as plsc

N = 10000
D = 256
NB = 10
E = 160000

NC, NS, L = 2, 16, 16

CONV_B = 128
GB = 64
GNB = 162
EPT = GNB * GB
EP = NS * EPT
ACC_ROWS = 10112
ROWS_PT = ACC_ROWS // NS

DEG_NB = 40
EPT_D = DEG_NB * CONV_B
EP_D = NC * NS * EPT_D
DACC = 10240
DROWS_PT = DACC // NS

B_IDX = 1024

PACK_SH = 14
PACK_MASK = (1 << PACK_SH) - 1


_MESH = plsc.VectorSubcoreMesh(core_axis_name="c", subcore_axis_name="s",
                               num_cores=NC, num_subcores=NS)


@functools.partial(
    pl.kernel,
    out_type=jax.ShapeDtypeStruct((NC, DACC), jnp.float32),
    mesh=_MESH,
    scratch_types=[
        pltpu.VMEM_SHARED((DACC,), jnp.float32),
        pltpu.VMEM((DEG_NB, CONV_B), jnp.int32),
        pltpu.VMEM((DROWS_PT,), jnp.float32),
        pltpu.VMEM((CONV_B,), jnp.float32),
    ],
)
def _sc_deg(dst_hbm, degp_hbm, acc, dd, zb, ones):
    c = lax.axis_index("c")
    s = lax.axis_index("s")

    def zfill(i, _):
        zb[pl.ds(i * L, L)] = jnp.zeros((L,), jnp.float32)
        return 0

    lax.fori_loop(0, DROWS_PT // L, zfill, 0)
    for jj in range(CONV_B // L):
        ones[pl.ds(jj * L, L)] = jnp.ones((L,), jnp.float32)
    pltpu.sync_copy(zb, acc.at[pl.ds(s * DROWS_PT, DROWS_PT)])
    pltpu.sync_copy(dst_hbm.at[c, s], dd)
    plsc.subcore_barrier()

    def body(j, _):
        pltpu.sync_copy(ones, acc.at[dd.at[j]], add=True)
        return 0

    lax.fori_loop(0, DEG_NB, body, 0)
    plsc.subcore_barrier()
    pltpu.sync_copy(acc.at[pl.ds(s * DROWS_PT, DROWS_PT)],
                    degp_hbm.at[c, pl.ds(s * DROWS_PT, DROWS_PT)])


@functools.partial(
    pl.kernel,
    out_type=jax.ShapeDtypeStruct((NC, ACC_ROWS, 128), jnp.float32),
    mesh=_MESH,
    scratch_types=[
        pltpu.VMEM_SHARED((ACC_ROWS, 128), jnp.float32),
        pltpu.VMEM((GNB, GB), jnp.int32),
        pltpu.VMEM((3, GB), jnp.int32),
        pltpu.VMEM((3, GB), jnp.int32),
        pltpu.VMEM((GB, 128), jnp.float32),
        pltpu.VMEM((GB, 128), jnp.float32),
        pltpu.VMEM((GB, 128), jnp.float32),
        pltpu.SemaphoreType.DMA,
        pltpu.SemaphoreType.DMA,
        pltpu.SemaphoreType.DMA,
    ],
)
def _sc_conv(y_hbm, pk_hbm, z_hbm, acc, pb, sb, db,
             r0, r1, r2, s0, s1, s2):
    c = lax.axis_index("c")
    s = lax.axis_index("s")
    bufs = (r0, r1, r2)
    sems = (s0, s1, s2)

    def zfill(i, _):
        for jj in range(128 // L):
            r0[i, pl.ds(jj * L, L)] = jnp.zeros((L,), jnp.float32)
        return 0

    lax.fori_loop(0, GB, zfill, 0)
    base = s * ROWS_PT
    for k in range(ROWS_PT // GB):
        pltpu.sync_copy(r0, acc.at[pl.ds(base + k * GB, GB)])
    pltpu.sync_copy(r0.at[pl.ds(0, ROWS_PT % GB)],
                    acc.at[pl.ds(base + (ROWS_PT // GB) * GB, ROWS_PT % GB)])
    pltpu.sync_copy(pk_hbm.at[c, s], pb)

    def unpack(j, t):
        for k in range(GB // L):
            v = pb[j, pl.ds(k * L, L)]
            db[t, pl.ds(k * L, L)] = lax.bitwise_and(v, PACK_MASK)
            sb[t, pl.ds(k * L, L)] = lax.shift_right_logical(v, PACK_SH)

    plsc.subcore_barrier()

    for t in range(3):
        unpack(t, t)
        pltpu.async_copy(y_hbm.at[sb.at[t]], bufs[t], sems[t])

    nb3 = GNB // 3

    def body3(j3, _):
        j = j3 * 3
        for t in range(3):
            pltpu.make_async_copy(y_hbm.at[sb.at[t]], bufs[t], sems[t]).wait()
            pltpu.sync_copy(bufs[t], acc.at[db.at[t]], add=True)

            @pl.when(j3 < nb3 - 1)
            def _reissue(t=t, bt=j + t):
                unpack(bt + 3, t)
                pltpu.async_copy(y_hbm.at[sb.at[t]], bufs[t], sems[t])

        return 0

    lax.fori_loop(0, nb3, body3, 0)

    plsc.subcore_barrier()
    for k in range(4):
        pltpu.sync_copy(acc.at[pl.ds(base + k * CONV_B, CONV_B)],
                        z_hbm.at[c, pl.ds(base + k * CONV_B, CONV_B)])
    pltpu.sync_copy(acc.at[pl.ds(base + 4 * CONV_B, ROWS_PT - 4 * CONV_B)],
                    z_hbm.at[c, pl.ds(base + 4 * CONV_B, ROWS_PT - 4 * CONV_B)])


@functools.partial(
    pl.kernel,
    out_type=jax.ShapeDtypeStruct((B_IDX, D), jnp.float32),
    mesh=_MESH,
    scratch_types=[
        pltpu.VMEM((B_IDX // (NC * NS),), jnp.int32),
        pltpu.VMEM((B_IDX // (NC * NS), D), jnp.float32),
        pltpu.SemaphoreType.DMA,
    ],
)
def _sc_gather(table_hbm, idx_hbm, out_hbm, idx_v, rows_v, sem):
    bpw = B_IDX // (NC * NS)
    wid = lax.axis_index("s") * NC + lax.axis_index("c")
    base = wid * bpw
    pltpu.sync_copy(idx_hbm.at[pl.ds(base, bpw)], idx_v)
    pltpu.async_copy(table_hbm.at[idx_v], rows_v, sem).wait()
    pltpu.sync_copy(rows_v, out_hbm.at[pl.ds(base, bpw)])


BR = 1000
GI = N // BR


def _dot(a, b):
    return jnp.dot(a, b, preferred_element_type=jnp.float32)


def _pre_body(x_ref, w_ref, dv_ref, o_ref):
    o_ref[...] = dv_ref[...] * _dot(x_ref[...], w_ref[...])


def _pre(x, w, dinv):
    return pl.pallas_call(
        _pre_body,
        grid=(GI, 2),
        in_specs=[
            pl.BlockSpec((BR, D), lambda i, j: (i, 0)),
            pl.BlockSpec((D, 128), lambda i, j: (0, j)),
            pl.BlockSpec((BR, 1), lambda i, j: (i, 0)),
        ],
        out_specs=pl.BlockSpec((BR, 128), lambda i, j: (j * GI + i, 0)),
        out_shape=jax.ShapeDtypeStruct((2 * N, 128), jnp.float32),
    )(x, w, dinv)


def _zy(z0_ref, z1_ref, y0_ref, y1_ref):
    zc = jnp.concatenate([z0_ref[0], z1_ref[0]], axis=1)
    yc = jnp.concatenate([y0_ref[...], y1_ref[...]], axis=1)
    return zc + yc


_ZY_SPECS = [
    pl.BlockSpec((1, BR, 128), lambda i, j: (0, i, 0)),
    pl.BlockSpec((1, BR, 128), lambda i, j: (1, i, 0)),
    pl.BlockSpec((BR, 128), lambda i, j: (i, 0)),
    pl.BlockSpec((BR, 128), lambda i, j: (GI + i, 0)),
]


def _postpre_body(act, emit_e, z0, z1, y0, y1, dv, b, wn, *outs):
    t = dv[...] * _zy(z0, z1, y0, y1) + b[...]
    if act == "relu":
        t = jnp.maximum(t, 0.0)
    if emit_e:
        outs[0][...] = t
    outs[-1][...] = dv[...] * _dot(t, wn[...])


def _postpre(z, y, dinv, b, wn, act=None, emit_e=False):
    out_shape = [jax.ShapeDtypeStruct((2 * N, 128), jnp.float32)]
    out_specs = [pl.BlockSpec((BR, 128), lambda i, j: (j * GI + i, 0))]
    if emit_e:
        out_shape.insert(0, jax.ShapeDtypeStruct((N, D), jnp.float32))
        out_specs.insert(0, pl.BlockSpec((BR, D), lambda i, j: (i, 0)))
    res = pl.pallas_call(
        functools.partial(_postpre_body, act, emit_e),
        grid=(GI, 2),
        in_specs=_ZY_SPECS + [
            pl.BlockSpec((BR, 1), lambda i, j: (i, 0)),
            pl.BlockSpec((1, D), lambda i, j: (0, 0)),
            pl.BlockSpec((D, 128), lambda i, j: (0, j)),
        ],
        out_specs=out_specs,
        out_shape=out_shape,
    )(z, z, y, y, dinv, b, wn)
    return res if emit_e else (res[0],)


def _postcond_body(z0, z1, y0, y1, dv, b, e1, w2, wi, bi, wh, bh, wo, bo,
                   ox, w0, yo_ref):
    e2 = dv[...] * _zy(z0, z1, y0, y1) + b[...] + e1[...]
    em = w2[0, 0] * e2
    h = _dot(em, wi[...]) + bi[...]
    h = jnp.where(h > 0, h, jnp.exp(jnp.minimum(h, 0.0)) - 1.0)
    h = _dot(h, wh[...]) + bh[...]
    h = jnp.where(h > 0, h, jnp.exp(jnp.minimum(h, 0.0)) - 1.0)
    prompt = _dot(h, wo[...]) + bo[...]
    xn = prompt * ox[...]
    yo_ref[...] = dv[...] * _dot(xn, w0[...])


def _postcond(z, y, dinv, b, e1, w2, layer, ox, w0):
    return pl.pallas_call(
        _postcond_body,
        grid=(GI, 2),
        in_specs=_ZY_SPECS + [
            pl.BlockSpec((BR, 1), lambda i, j: (i, 0)),
            pl.BlockSpec((1, D), lambda i, j: (0, 0)),
            pl.BlockSpec((BR, D), lambda i, j: (i, 0)),
            pl.BlockSpec((1, 1), lambda i, j: (0, 0)),
            pl.BlockSpec((D, D), lambda i, j: (0, 0)),
            pl.BlockSpec((1, D), lambda i, j: (0, 0)),
            pl.BlockSpec((D, D), lambda i, j: (0, 0)),
            pl.BlockSpec((1, D), lambda i, j: (0, 0)),
            pl.BlockSpec((D, D), lambda i, j: (0, 0)),
            pl.BlockSpec((1, D), lambda i, j: (0, 0)),
            pl.BlockSpec((BR, D), lambda i, j: (i, 0)),
            pl.BlockSpec((D, 128), lambda i, j: (0, j)),
        ],
        out_specs=pl.BlockSpec((BR, 128), lambda i, j: (j * GI + i, 0)),
        out_shape=jax.ShapeDtypeStruct((2 * N, 128), jnp.float32),
    )(z, z, y, y, dinv, b, e1, w2,
      layer["Wi"], layer["bi"].reshape(1, D),
      layer["Wh"], layer["bh"].reshape(1, D),
      layer["Wo"], layer["bo"].reshape(1, D), ox, w0)


def _posthead_body(z0, z1, y0, y1, dv, b, wa, ba, plist, o_ref):
    embed = dv[...] * _zy(z0, z1, y0, y1) + b[...]
    score = _dot(embed, wa[...]) + ba[...]
    m = jnp.max(score, axis=1, keepdims=True)
    ex = jnp.exp(score - m)
    weight = ex / jnp.sum(ex, axis=1, keepdims=True)
    o_ref[...] = embed + _dot(weight, plist[...])


def _posthead(z, y, dinv, b, wa, ba, plist):
    return pl.pallas_call(
        _posthead_body,
        grid=(GI,),
        in_specs=[
            pl.BlockSpec((1, BR, 128), lambda i: (0, i, 0)),
            pl.BlockSpec((1, BR, 128), lambda i: (1, i, 0)),
            pl.BlockSpec((BR, 128), lambda i: (i, 0)),
            pl.BlockSpec((BR, 128), lambda i: (GI + i, 0)),
            pl.BlockSpec((BR, 1), lambda i: (i, 0)),
            pl.BlockSpec((1, D), lambda i: (0, 0)),
            pl.BlockSpec((D, 5), lambda i: (0, 0)),
            pl.BlockSpec((1, 5), lambda i: (0, 0)),
            pl.BlockSpec((5, D), lambda i: (0, 0)),
        ],
        out_specs=pl.BlockSpec((BR, D), lambda i: (i, 0)),
        out_shape=jax.ShapeDtypeStruct((N, D), jnp.float32),
    )(z, z, y, y, dinv, b, wa, ba, plist)


def _head_body(raw_ref, lab_ref, gate_ref, o_ref):
    raw = raw_ref[...]
    oh = (lax.broadcasted_iota(jnp.int32, (NB, B_IDX), 0)
          == lab_ref[...]).astype(jnp.float32)
    sums = _dot(oh, raw)
    cnts = jnp.sum(oh, axis=1, keepdims=True)
    ave = sums / jnp.maximum(cnts, 1.0) * gate_ref[0, 0]
    r = raw[0:1000, :]
    rn = jnp.sqrt(jnp.sum(r * r, axis=1, keepdims=True))
    an_sq = lax.dot_general(jnp.ones((1, D), jnp.float32), ave * ave,
                            (((1,), (1,)), ((), ())),
                            preferred_element_type=jnp.float32)
    dots = lax.dot_general(r, ave, (((1,), (1,)), ((), ())),
                           preferred_element_type=jnp.float32)
    denom = jnp.maximum(rn * jnp.sqrt(an_sq), 1e-8)
    ret = dots / denom
    m = jnp.max(ret, axis=1, keepdims=True)
    ex = jnp.exp(ret - m)
    o_ref[...] = ex / jnp.sum(ex, axis=1, keepdims=True)


def _head(raw, labels_pad, gate):
    return pl.pallas_call(
        _head_body,
        out_shape=jax.ShapeDtypeStruct((1000, NB), jnp.float32),
    )(raw, labels_pad, gate)


def kernel(x, params, edge_index, idx, labels, train):
    src = edge_index[0]
    dst = edge_index[1]

    pad = EP - E
    src_p = jnp.concatenate([src, jnp.zeros((pad,), jnp.int32)])
    dst_p = jnp.concatenate(
        [dst, N + (jnp.arange(pad, dtype=jnp.int32) % (ACC_ROWS - N))])
    src_r = src_p.reshape(NS, GNB, GB)
    src3 = jnp.stack([src_r, src_r + N])
    dst3 = dst_p.reshape(NS, GNB, GB)
    pk3 = src3 * (1 << PACK_SH) + dst3[None]

    pad_d = EP_D - E
    dst_dp = jnp.concatenate(
        [dst, N + (jnp.arange(pad_d, dtype=jnp.int32) % (DACC - N))])
    dstd = dst_dp.reshape(NC, NS, DEG_NB, CONV_B)

    degp = _sc_deg(dstd)
    deg = degp[0, :N] + degp[1, :N] + 1.0
    dinv = lax.rsqrt(deg).reshape(N, 1)

    Wg = params["gcn_W"]
    bg = [b.reshape(1, D) for b in params["gcn_b"]]
    w2 = params["gcn_weight2"].reshape(1, 1)

    ox = x
    y = _pre(x, Wg[0], dinv)
    for layer in params["cond"]:
        z = _sc_conv(y, pk3)
        e1, y = _postpre(z, y, dinv, bg[0], Wg[1], emit_e=True)
        z = _sc_conv(y, pk3)
        y = _postcond(z, y, dinv, bg[1], e1, w2, layer, ox, Wg[0])
    z = _sc_conv(y, pk3)
    (y,) = _postpre(z, y, dinv, bg[0], Wg[1], act="relu")
    z = _sc_conv(y, pk3)
    (y,) = _postpre(z, y, dinv, bg[1], Wg[2], act="relu")
    z = _sc_conv(y, pk3)
    embed = _posthead(z, y, dinv, bg[2], params["Wa"],
                      params["ba"].reshape(1, 5), params["p_list"])

    idx_p = jnp.concatenate([idx, jnp.zeros((B_IDX - 1000,), jnp.int32)])
    raw = _sc_gather(embed, idx_p)
    labels_p = jnp.concatenate(
        [labels, jnp.full((B_IDX - 1000,), NB, jnp.int32)]).reshape(1, B_IDX)
    gate = jnp.where(train == 1, 1.0, 0.0).astype(jnp.float32).reshape(1, 1)
    return _head(raw, labels_p, gate)


# speedup vs baseline: 3.1128x; 3.1128x over previous
import functools

import jax
import jax.numpy as jnp
from jax import lax
from jax.experimental import pallas as pl
from jax.experimental.pallas import tpu as pltpu
from jax.experimental.pallas import tpu_sc as plsc

N = 10000
D = 256
NB = 10
E = 160000

NC, NS, L = 2, 16, 16

CONV_B = 128
CONV_NB = 79
EPT = CONV_NB * CONV_B
EP = NS * EPT
ACC_ROWS = 10112
ROWS_PT = ACC_ROWS // NS

DEG_NB = 40
EPT_D = DEG_NB * CONV_B
EP_D = NC * NS * EPT_D
DACC = 10240
DROWS_PT = DACC // NS

B_IDX = 1024

PACK_SH = 14
PACK_MASK = (1 << PACK_SH) - 1


_MESH = plsc.VectorSubcoreMesh(core_axis_name="c", subcore_axis_name="s",
                               num_cores=NC, num_subcores=NS)


@functools.partial(
    pl.kernel,
    out_type=jax.ShapeDtypeStruct((NC, DACC), jnp.float32),
    mesh=_MESH,
    scratch_types=[
        pltpu.VMEM_SHARED((DACC,), jnp.float32),
        pltpu.VMEM((DEG_NB, CONV_B), jnp.int32),
        pltpu.VMEM((DROWS_PT,), jnp.float32),
        pltpu.VMEM((CONV_B,), jnp.float32),
    ],
)
def _sc_deg(dst_hbm, degp_hbm, acc, dd, zb, ones):
    c = lax.axis_index("c")
    s = lax.axis_index("s")

    def zfill(i, _):
        zb[pl.ds(i * L, L)] = jnp.zeros((L,), jnp.float32)
        return 0

    lax.fori_loop(0, DROWS_PT // L, zfill, 0)
    for jj in range(CONV_B // L):
        ones[pl.ds(jj * L, L)] = jnp.ones((L,), jnp.float32)
    pltpu.sync_copy(zb, acc.at[pl.ds(s * DROWS_PT, DROWS_PT)])
    pltpu.sync_copy(dst_hbm.at[c, s], dd)
    plsc.subcore_barrier()

    def body(j, _):
        pltpu.sync_copy(ones, acc.at[dd.at[j]], add=True)
        return 0

    lax.fori_loop(0, DEG_NB, body, 0)
    plsc.subcore_barrier()
    pltpu.sync_copy(acc.at[pl.ds(s * DROWS_PT, DROWS_PT)],
                    degp_hbm.at[c, pl.ds(s * DROWS_PT, DROWS_PT)])


@functools.partial(
    pl.kernel,
    out_type=jax.ShapeDtypeStruct((NC, ACC_ROWS, 128), jnp.float32),
    mesh=_MESH,
    scratch_types=[
        pltpu.VMEM_SHARED((ACC_ROWS, 128), jnp.float32),
        pltpu.VMEM((CONV_NB, CONV_B), jnp.int32),
        pltpu.VMEM((2, CONV_B), jnp.int32),
        pltpu.VMEM((2, CONV_B), jnp.int32),
        pltpu.VMEM((CONV_B, 128), jnp.float32),
        pltpu.VMEM((CONV_B, 128), jnp.float32),
        pltpu.SemaphoreType.DMA,
        pltpu.SemaphoreType.DMA,
    ],
)
def _sc_conv(y_hbm, pk_hbm, z_hbm, acc, pb, sb, db, ra, rb, sema, semb):
    c = lax.axis_index("c")
    s = lax.axis_index("s")

    def zfill(i, _):
        for jj in range(128 // L):
            ra[i, pl.ds(jj * L, L)] = jnp.zeros((L,), jnp.float32)
        return 0

    lax.fori_loop(0, CONV_B, zfill, 0)
    base = s * ROWS_PT
    for k in range(4):
        pltpu.sync_copy(ra, acc.at[pl.ds(base + k * CONV_B, CONV_B)])
    pltpu.sync_copy(ra.at[pl.ds(0, ROWS_PT - 4 * CONV_B)],
                    acc.at[pl.ds(base + 4 * CONV_B, ROWS_PT - 4 * CONV_B)])
    pltpu.sync_copy(pk_hbm.at[c, s], pb)

    def unpack(j, t):
        for k in range(CONV_B // L):
            v = pb[j, pl.ds(k * L, L)]
            db[t, pl.ds(k * L, L)] = lax.bitwise_and(v, PACK_MASK)
            sb[t, pl.ds(k * L, L)] = lax.shift_right_logical(v, PACK_SH)

    plsc.subcore_barrier()

    unpack(0, 0)
    pltpu.async_copy(y_hbm.at[sb.at[0]], ra, sema)

    def body2(j2, _):
        j = j2 * 2
        unpack(j + 1, 1)
        pltpu.async_copy(y_hbm.at[sb.at[1]], rb, semb)
        pltpu.make_async_copy(y_hbm.at[sb.at[0]], ra, sema).wait()
        pltpu.sync_copy(ra, acc.at[db.at[0]], add=True)
        unpack(j + 2, 0)
        pltpu.async_copy(y_hbm.at[sb.at[0]], ra, sema)
        pltpu.make_async_copy(y_hbm.at[sb.at[1]], rb, semb).wait()
        pltpu.sync_copy(rb, acc.at[db.at[1]], add=True)
        return 0

    lax.fori_loop(0, (CONV_NB - 1) // 2, body2, 0)
    pltpu.make_async_copy(y_hbm.at[sb.at[0]], ra, sema).wait()
    pltpu.sync_copy(ra, acc.at[db.at[0]], add=True)

    plsc.subcore_barrier()
    for k in range(4):
        pltpu.sync_copy(acc.at[pl.ds(base + k * CONV_B, CONV_B)],
                        z_hbm.at[c, pl.ds(base + k * CONV_B, CONV_B)])
    pltpu.sync_copy(acc.at[pl.ds(base + 4 * CONV_B, ROWS_PT - 4 * CONV_B)],
                    z_hbm.at[c, pl.ds(base + 4 * CONV_B, ROWS_PT - 4 * CONV_B)])


@functools.partial(
    pl.kernel,
    out_type=jax.ShapeDtypeStruct((B_IDX, D), jnp.float32),
    mesh=_MESH,
    scratch_types=[
        pltpu.VMEM((B_IDX // (NC * NS),), jnp.int32),
        pltpu.VMEM((B_IDX // (NC * NS), D), jnp.float32),
        pltpu.SemaphoreType.DMA,
    ],
)
def _sc_gather(table_hbm, idx_hbm, out_hbm, idx_v, rows_v, sem):
    bpw = B_IDX // (NC * NS)
    wid = lax.axis_index("s") * NC + lax.axis_index("c")
    base = wid * bpw
    pltpu.sync_copy(idx_hbm.at[pl.ds(base, bpw)], idx_v)
    pltpu.async_copy(table_hbm.at[idx_v], rows_v, sem).wait()
    pltpu.sync_copy(rows_v, out_hbm.at[pl.ds(base, bpw)])


BR = 1000
GI = N // BR


def _dot(a, b):
    return jnp.dot(a, b, preferred_element_type=jnp.float32)


def _pre_body(x_ref, w_ref, dv_ref, o_ref):
    o_ref[...] = dv_ref[...] * _dot(x_ref[...], w_ref[...])


def _pre(x, w, dinv):
    return pl.pallas_call(
        _pre_body,
        grid=(GI, 2),
        in_specs=[
            pl.BlockSpec((BR, D), lambda i, j: (i, 0)),
            pl.BlockSpec((D, 128), lambda i, j: (0, j)),
            pl.BlockSpec((BR, 1), lambda i, j: (i, 0)),
        ],
        out_specs=pl.BlockSpec((BR, 128), lambda i, j: (j * GI + i, 0)),
        out_shape=jax.ShapeDtypeStruct((2 * N, 128), jnp.float32),
    )(x, w, dinv)


def _zy(z0_ref, z1_ref, y0_ref, y1_ref):
    zc = jnp.concatenate([z0_ref[0], z1_ref[0]], axis=1)
    yc = jnp.concatenate([y0_ref[...], y1_ref[...]], axis=1)
    return zc + yc


_ZY_SPECS = [
    pl.BlockSpec((1, BR, 128), lambda i, j: (0, i, 0)),
    pl.BlockSpec((1, BR, 128), lambda i, j: (1, i, 0)),
    pl.BlockSpec((BR, 128), lambda i, j: (i, 0)),
    pl.BlockSpec((BR, 128), lambda i, j: (GI + i, 0)),
]


def _postpre_body(act, emit_e, z0, z1, y0, y1, dv, b, wn, *outs):
    t = dv[...] * _zy(z0, z1, y0, y1) + b[...]
    if act == "relu":
        t = jnp.maximum(t, 0.0)
    if emit_e:
        outs[0][...] = t
    outs[-1][...] = dv[...] * _dot(t, wn[...])


def _postpre(z, y, dinv, b, wn, act=None, emit_e=False):
    out_shape = [jax.ShapeDtypeStruct((2 * N, 128), jnp.float32)]
    out_specs = [pl.BlockSpec((BR, 128), lambda i, j: (j * GI + i, 0))]
    if emit_e:
        out_shape.insert(0, jax.ShapeDtypeStruct((N, D), jnp.float32))
        out_specs.insert(0, pl.BlockSpec((BR, D), lambda i, j: (i, 0)))
    res = pl.pallas_call(
        functools.partial(_postpre_body, act, emit_e),
        grid=(GI, 2),
        in_specs=_ZY_SPECS + [
            pl.BlockSpec((BR, 1), lambda i, j: (i, 0)),
            pl.BlockSpec((1, D), lambda i, j: (0, 0)),
            pl.BlockSpec((D, 128), lambda i, j: (0, j)),
        ],
        out_specs=out_specs,
        out_shape=out_shape,
    )(z, z, y, y, dinv, b, wn)
    return res if emit_e else (res[0],)


def _postcond_body(z0, z1, y0, y1, dv, b, e1, w2, wi, bi, wh, bh, wo, bo,
                   ox, w0, yo_ref):
    e2 = dv[...] * _zy(z0, z1, y0, y1) + b[...] + e1[...]
    em = w2[0, 0] * e2
    h = _dot(em, wi[...]) + bi[...]
    h = jnp.where(h > 0, h, jnp.exp(jnp.minimum(h, 0.0)) - 1.0)
    h = _dot(h, wh[...]) + bh[...]
    h = jnp.where(h > 0, h, jnp.exp(jnp.minimum(h, 0.0)) - 1.0)
    prompt = _dot(h, wo[...]) + bo[...]
    xn = prompt * ox[...]
    yo_ref[...] = dv[...] * _dot(xn, w0[...])


def _postcond(z, y, dinv, b, e1, w2, layer, ox, w0):
    return pl.pallas_call(
        _postcond_body,
        grid=(GI, 2),
        in_specs=_ZY_SPECS + [
            pl.BlockSpec((BR, 1), lambda i, j: (i, 0)),
            pl.BlockSpec((1, D), lambda i, j: (0, 0)),
            pl.BlockSpec((BR, D), lambda i, j: (i, 0)),
            pl.BlockSpec((1, 1), lambda i, j: (0, 0)),
            pl.BlockSpec((D, D), lambda i, j: (0, 0)),
            pl.BlockSpec((1, D), lambda i, j: (0, 0)),
            pl.BlockSpec((D, D), lambda i, j: (0, 0)),
            pl.BlockSpec((1, D), lambda i, j: (0, 0)),
            pl.BlockSpec((D, D), lambda i, j: (0, 0)),
            pl.BlockSpec((1, D), lambda i, j: (0, 0)),
            pl.BlockSpec((BR, D), lambda i, j: (i, 0)),
            pl.BlockSpec((D, 128), lambda i, j: (0, j)),
        ],
        out_specs=pl.BlockSpec((BR, 128), lambda i, j: (j * GI + i, 0)),
        out_shape=jax.ShapeDtypeStruct((2 * N, 128), jnp.float32),
    )(z, z, y, y, dinv, b, e1, w2,
      layer["Wi"], layer["bi"].reshape(1, D),
      layer["Wh"], layer["bh"].reshape(1, D),
      layer["Wo"], layer["bo"].reshape(1, D), ox, w0)


def _posthead_body(z0, z1, y0, y1, dv, b, wa, ba, plist, o_ref):
    embed = dv[...] * _zy(z0, z1, y0, y1) + b[...]
    score = _dot(embed, wa[...]) + ba[...]
    m = jnp.max(score, axis=1, keepdims=True)
    ex = jnp.exp(score - m)
    weight = ex / jnp.sum(ex, axis=1, keepdims=True)
    o_ref[...] = embed + _dot(weight, plist[...])


def _posthead(z, y, dinv, b, wa, ba, plist):
    return pl.pallas_call(
        _posthead_body,
        grid=(GI,),
        in_specs=[
            pl.BlockSpec((1, BR, 128), lambda i: (0, i, 0)),
            pl.BlockSpec((1, BR, 128), lambda i: (1, i, 0)),
            pl.BlockSpec((BR, 128), lambda i: (i, 0)),
            pl.BlockSpec((BR, 128), lambda i: (GI + i, 0)),
            pl.BlockSpec((BR, 1), lambda i: (i, 0)),
            pl.BlockSpec((1, D), lambda i: (0, 0)),
            pl.BlockSpec((D, 5), lambda i: (0, 0)),
            pl.BlockSpec((1, 5), lambda i: (0, 0)),
            pl.BlockSpec((5, D), lambda i: (0, 0)),
        ],
        out_specs=pl.BlockSpec((BR, D), lambda i: (i, 0)),
        out_shape=jax.ShapeDtypeStruct((N, D), jnp.float32),
    )(z, z, y, y, dinv, b, wa, ba, plist)


def _head_body(raw_ref, lab_ref, gate_ref, o_ref):
    raw = raw_ref[...]
    oh = (lax.broadcasted_iota(jnp.int32, (NB, B_IDX), 0)
          == lab_ref[...]).astype(jnp.float32)
    sums = _dot(oh, raw)
    cnts = jnp.sum(oh, axis=1, keepdims=True)
    ave = sums / jnp.maximum(cnts, 1.0) * gate_ref[0, 0]
    r = raw[0:1000, :]
    rn = jnp.sqrt(jnp.sum(r * r, axis=1, keepdims=True))
    an_sq = lax.dot_general(jnp.ones((1, D), jnp.float32), ave * ave,
                            (((1,), (1,)), ((), ())),
                            preferred_element_type=jnp.float32)
    dots = lax.dot_general(r, ave, (((1,), (1,)), ((), ())),
                           preferred_element_type=jnp.float32)
    denom = jnp.maximum(rn * jnp.sqrt(an_sq), 1e-8)
    ret = dots / denom
    m = jnp.max(ret, axis=1, keepdims=True)
    ex = jnp.exp(ret - m)
    o_ref[...] = ex / jnp.sum(ex, axis=1, keepdims=True)


def _head(raw, labels_pad, gate):
    return pl.pallas_call(
        _head_body,
        out_shape=jax.ShapeDtypeStruct((1000, NB), jnp.float32),
    )(raw, labels_pad, gate)


def kernel(x, params, edge_index, idx, labels, train):
    src = edge_index[0]
    dst = edge_index[1]

    pad = EP - E
    src_p = jnp.concatenate(
        [src, jnp.arange(pad, dtype=jnp.int32) % 128])
    dst_p = jnp.concatenate(
        [dst, N + (jnp.arange(pad, dtype=jnp.int32) % (ACC_ROWS - N))])
    src_r = src_p.reshape(NS, CONV_NB, CONV_B)
    src3 = jnp.stack([src_r, src_r + N])
    dst3 = dst_p.reshape(NS, CONV_NB, CONV_B)
    pk3 = src3 * (1 << PACK_SH) + dst3[None]

    pad_d = EP_D - E
    dst_dp = jnp.concatenate(
        [dst, N + (jnp.arange(pad_d, dtype=jnp.int32) % (DACC - N))])
    dstd = dst_dp.reshape(NC, NS, DEG_NB, CONV_B)

    degp = _sc_deg(dstd)
    deg = degp[0, :N] + degp[1, :N] + 1.0
    dinv = lax.rsqrt(deg).reshape(N, 1)

    Wg = params["gcn_W"]
    bg = [b.reshape(1, D) for b in params["gcn_b"]]
    w2 = params["gcn_weight2"].reshape(1, 1)

    ox = x
    y = _pre(x, Wg[0], dinv)
    for layer in params["cond"]:
        z = _sc_conv(y, pk3)
        e1, y = _postpre(z, y, dinv, bg[0], Wg[1], emit_e=True)
        z = _sc_conv(y, pk3)
        y = _postcond(z, y, dinv, bg[1], e1, w2, layer, ox, Wg[0])
    z = _sc_conv(y, pk3)
    (y,) = _postpre(z, y, dinv, bg[0], Wg[1], act="relu")
    z = _sc_conv(y, pk3)
    (y,) = _postpre(z, y, dinv, bg[1], Wg[2], act="relu")
    z = _sc_conv(y, pk3)
    embed = _posthead(z, y, dinv, bg[2], params["Wa"],
                      params["ba"].reshape(1, 5), params["p_list"])

    idx_p = jnp.concatenate([idx, jnp.zeros((B_IDX - 1000,), jnp.int32)])
    raw = _sc_gather(embed, idx_p)
    labels_p = jnp.concatenate(
        [labels, jnp.full((B_IDX - 1000,), NB, jnp.int32)]).reshape(1, B_IDX)
    gate = jnp.where(train == 1, 1.0, 0.0).astype(jnp.float32).reshape(1, 1)
    return _head(raw, labels_p, gate)


# speedup vs baseline: 3.1135x; 1.0002x over previous
import functools

import jax
import jax.numpy as jnp
from jax import lax
from jax.experimental import pallas as pl
from jax.experimental.pallas import tpu as pltpu
from jax.experimental.pallas import tpu_sc as plsc

N = 10000
D = 256
NB = 10
E = 160000

NC, NS, L = 2, 16, 16

CONV_B = 128
CONV_NB = 79
EPT = CONV_NB * CONV_B
EP = NS * EPT
ACC_ROWS = 10112
ROWS_PT = ACC_ROWS // NS

DEG_NB = 40
EPT_D = DEG_NB * CONV_B
EP_D = NC * NS * EPT_D
DACC = 10240
DROWS_PT = DACC // NS

B_IDX = 1024

PACK_SH = 14
PACK_MASK = (1 << PACK_SH) - 1


_MESH = plsc.VectorSubcoreMesh(core_axis_name="c", subcore_axis_name="s",
                               num_cores=NC, num_subcores=NS)


@functools.partial(
    pl.kernel,
    out_type=jax.ShapeDtypeStruct((NC, DACC), jnp.float32),
    mesh=_MESH,
    scratch_types=[
        pltpu.VMEM_SHARED((DACC,), jnp.float32),
        pltpu.VMEM((DEG_NB, CONV_B), jnp.int32),
        pltpu.VMEM((DROWS_PT,), jnp.float32),
        pltpu.VMEM((CONV_B,), jnp.float32),
    ],
)
def _sc_deg(dst_hbm, degp_hbm, acc, dd, zb, ones):
    c = lax.axis_index("c")
    s = lax.axis_index("s")

    def zfill(i, _):
        zb[pl.ds(i * L, L)] = jnp.zeros((L,), jnp.float32)
        return 0

    lax.fori_loop(0, DROWS_PT // L, zfill, 0)
    for jj in range(CONV_B // L):
        ones[pl.ds(jj * L, L)] = jnp.ones((L,), jnp.float32)
    pltpu.sync_copy(zb, acc.at[pl.ds(s * DROWS_PT, DROWS_PT)])
    pltpu.sync_copy(dst_hbm.at[c, s], dd)
    plsc.subcore_barrier()

    def body(j, _):
        pltpu.sync_copy(ones, acc.at[dd.at[j]], add=True)
        return 0

    lax.fori_loop(0, DEG_NB, body, 0)
    plsc.subcore_barrier()
    pltpu.sync_copy(acc.at[pl.ds(s * DROWS_PT, DROWS_PT)],
                    degp_hbm.at[c, pl.ds(s * DROWS_PT, DROWS_PT)])


@functools.partial(
    pl.kernel,
    out_type=jax.ShapeDtypeStruct((NC, ACC_ROWS, 128), jnp.float32),
    mesh=_MESH,
    scratch_types=[
        pltpu.VMEM_SHARED((ACC_ROWS, 128), jnp.float32),
        pltpu.VMEM((CONV_NB, CONV_B), jnp.int32),
        pltpu.VMEM((2, CONV_B), jnp.int32),
        pltpu.VMEM((2, CONV_B), jnp.int32),
        pltpu.VMEM((CONV_B, 128), jnp.float32),
        pltpu.VMEM((CONV_B, 128), jnp.float32),
        pltpu.SemaphoreType.DMA,
        pltpu.SemaphoreType.DMA,
    ],
)
def _sc_conv(y_hbm, pk_hbm, z_hbm, acc, pb, sb, db, ra, rb, sema, semb):
    c = lax.axis_index("c")
    s = lax.axis_index("s")

    def zfill(i, _):
        for jj in range(128 // L):
            ra[i, pl.ds(jj * L, L)] = jnp.zeros((L,), jnp.float32)
        return 0

    lax.fori_loop(0, CONV_B, zfill, 0)
    base = s * ROWS_PT
    for k in range(4):
        pltpu.sync_copy(ra, acc.at[pl.ds(base + k * CONV_B, CONV_B)])
    pltpu.sync_copy(ra.at[pl.ds(0, ROWS_PT - 4 * CONV_B)],
                    acc.at[pl.ds(base + 4 * CONV_B, ROWS_PT - 4 * CONV_B)])
    pltpu.sync_copy(pk_hbm.at[c, s], pb)

    def unpack(j, t):
        for k in range(CONV_B // L):
            v = pb[j, pl.ds(k * L, L)]
            db[t, pl.ds(k * L, L)] = lax.bitwise_and(v, PACK_MASK)
            sb[t, pl.ds(k * L, L)] = lax.shift_right_logical(v, PACK_SH)

    plsc.subcore_barrier()

    unpack(0, 0)
    pltpu.async_copy(y_hbm.at[sb.at[0]], ra, sema)

    def body2(j2, _):
        j = j2 * 2
        unpack(j + 1, 1)
        pltpu.async_copy(y_hbm.at[sb.at[1]], rb, semb)
        pltpu.make_async_copy(y_hbm.at[sb.at[0]], ra, sema).wait()
        pltpu.sync_copy(ra, acc.at[db.at[0]], add=True)
        unpack(j + 2, 0)
        pltpu.async_copy(y_hbm.at[sb.at[0]], ra, sema)
        pltpu.make_async_copy(y_hbm.at[sb.at[1]], rb, semb).wait()
        pltpu.sync_copy(rb, acc.at[db.at[1]], add=True)
        return 0

    lax.fori_loop(0, (CONV_NB - 1) // 2, body2, 0)
    pltpu.make_async_copy(y_hbm.at[sb.at[0]], ra, sema).wait()
    pltpu.sync_copy(ra, acc.at[db.at[0]], add=True)

    plsc.subcore_barrier()
    for k in range(4):
        pltpu.sync_copy(acc.at[pl.ds(base + k * CONV_B, CONV_B)],
                        z_hbm.at[c, pl.ds(base + k * CONV_B, CONV_B)])
    pltpu.sync_copy(acc.at[pl.ds(base + 4 * CONV_B, ROWS_PT - 4 * CONV_B)],
                    z_hbm.at[c, pl.ds(base + 4 * CONV_B, ROWS_PT - 4 * CONV_B)])


@functools.partial(
    pl.kernel,
    out_type=jax.ShapeDtypeStruct((B_IDX, D), jnp.float32),
    mesh=_MESH,
    scratch_types=[
        pltpu.VMEM((B_IDX // (NC * NS),), jnp.int32),
        pltpu.VMEM((B_IDX // (NC * NS), D), jnp.float32),
        pltpu.SemaphoreType.DMA,
    ],
)
def _sc_gather(table_hbm, idx_hbm, out_hbm, idx_v, rows_v, sem):
    bpw = B_IDX // (NC * NS)
    wid = lax.axis_index("s") * NC + lax.axis_index("c")
    base = wid * bpw
    pltpu.sync_copy(idx_hbm.at[pl.ds(base, bpw)], idx_v)
    pltpu.async_copy(table_hbm.at[idx_v], rows_v, sem).wait()
    pltpu.sync_copy(rows_v, out_hbm.at[pl.ds(base, bpw)])


BR = 1000
GI = N // BR


def _dot(a, b):
    return jnp.dot(a, b, preferred_element_type=jnp.float32)


def _pre_body(x_ref, w_ref, dv_ref, o_ref):
    o_ref[...] = dv_ref[...] * _dot(x_ref[...], w_ref[...])


def _pre(x, w, dinv):
    return pl.pallas_call(
        _pre_body,
        grid=(GI, 2),
        in_specs=[
            pl.BlockSpec((BR, D), lambda i, j: (i, 0)),
            pl.BlockSpec((D, 128), lambda i, j: (0, j)),
            pl.BlockSpec((BR, 1), lambda i, j: (i, 0)),
        ],
        out_specs=pl.BlockSpec((BR, 128), lambda i, j: (j * GI + i, 0)),
        out_shape=jax.ShapeDtypeStruct((2 * N, 128), jnp.float32),
    )(x, w, dinv)


def _zy(z0_ref, z1_ref, y0_ref, y1_ref):
    zc = jnp.concatenate([z0_ref[0], z1_ref[0]], axis=1)
    yc = jnp.concatenate([y0_ref[...], y1_ref[...]], axis=1)
    return zc + yc


_ZY_SPECS = [
    pl.BlockSpec((1, BR, 128), lambda i, j: (0, i, 0)),
    pl.BlockSpec((1, BR, 128), lambda i, j: (1, i, 0)),
    pl.BlockSpec((BR, 128), lambda i, j: (i, 0)),
    pl.BlockSpec((BR, 128), lambda i, j: (GI + i, 0)),
]


def _postpre_body(act, emit_e, z0, z1, y0, y1, dv, b, wn, *outs):
    t = dv[...] * _zy(z0, z1, y0, y1) + b[...]
    if act == "relu":
        t = jnp.maximum(t, 0.0)
    if emit_e:
        outs[0][...] = t
    outs[-1][...] = dv[...] * _dot(t, wn[...])


def _postpre(z, y, dinv, b, wn, act=None, emit_e=False):
    out_shape = [jax.ShapeDtypeStruct((2 * N, 128), jnp.float32)]
    out_specs = [pl.BlockSpec((BR, 128), lambda i, j: (j * GI + i, 0))]
    if emit_e:
        out_shape.insert(0, jax.ShapeDtypeStruct((N, D), jnp.float32))
        out_specs.insert(0, pl.BlockSpec((BR, D), lambda i, j: (i, 0)))
    res = pl.pallas_call(
        functools.partial(_postpre_body, act, emit_e),
        grid=(GI, 2),
        in_specs=_ZY_SPECS + [
            pl.BlockSpec((BR, 1), lambda i, j: (i, 0)),
            pl.BlockSpec((1, D), lambda i, j: (0, 0)),
            pl.BlockSpec((D, 128), lambda i, j: (0, j)),
        ],
        out_specs=out_specs,
        out_shape=out_shape,
    )(z, z, y, y, dinv, b, wn)
    return res if emit_e else (res[0],)


def _postcond_body(z0, z1, y0, y1, dv, b, e1, w2, wi, bi, wh, bh, wo, bo,
                   ox, w0, yo_ref):
    e2 = dv[...] * _zy(z0, z1, y0, y1) + b[...] + e1[...]
    em = w2[0, 0] * e2
    h = _dot(em, wi[...]) + bi[...]
    h = jnp.where(h > 0, h, jnp.exp(jnp.minimum(h, 0.0)) - 1.0)
    h = _dot(h, wh[...]) + bh[...]
    h = jnp.where(h > 0, h, jnp.exp(jnp.minimum(h, 0.0)) - 1.0)
    prompt = _dot(h, wo[...]) + bo[...]
    xn = prompt * ox[...]
    yo_ref[...] = dv[...] * _dot(xn, w0[...])


def _postcond(z, y, dinv, b, e1, w2, layer, ox, w0):
    return pl.pallas_call(
        _postcond_body,
        grid=(GI, 2),
        in_specs=_ZY_SPECS + [
            pl.BlockSpec((BR, 1), lambda i, j: (i, 0)),
            pl.BlockSpec((1, D), lambda i, j: (0, 0)),
            pl.BlockSpec((BR, D), lambda i, j: (i, 0)),
            pl.BlockSpec((1, 1), lambda i, j: (0, 0)),
            pl.BlockSpec((D, D), lambda i, j: (0, 0)),
            pl.BlockSpec((1, D), lambda i, j: (0, 0)),
            pl.BlockSpec((D, D), lambda i, j: (0, 0)),
            pl.BlockSpec((1, D), lambda i, j: (0, 0)),
            pl.BlockSpec((D, D), lambda i, j: (0, 0)),
            pl.BlockSpec((1, D), lambda i, j: (0, 0)),
            pl.BlockSpec((BR, D), lambda i, j: (i, 0)),
            pl.BlockSpec((D, 128), lambda i, j: (0, j)),
        ],
        out_specs=pl.BlockSpec((BR, 128), lambda i, j: (j * GI + i, 0)),
        out_shape=jax.ShapeDtypeStruct((2 * N, 128), jnp.float32),
    )(z, z, y, y, dinv, b, e1, w2,
      layer["Wi"], layer["bi"].reshape(1, D),
      layer["Wh"], layer["bh"].reshape(1, D),
      layer["Wo"], layer["bo"].reshape(1, D), ox, w0)


def _posthead_body(z0, z1, y0, y1, dv, b, wa, ba, plist, o_ref):
    embed = dv[...] * _zy(z0, z1, y0, y1) + b[...]
    score = _dot(embed, wa[...]) + ba[...]
    m = jnp.max(score, axis=1, keepdims=True)
    ex = jnp.exp(score - m)
    weight = ex / jnp.sum(ex, axis=1, keepdims=True)
    o_ref[...] = embed + _dot(weight, plist[...])


def _posthead(z, y, dinv, b, wa, ba, plist):
    return pl.pallas_call(
        _posthead_body,
        grid=(GI,),
        in_specs=[
            pl.BlockSpec((1, BR, 128), lambda i: (0, i, 0)),
            pl.BlockSpec((1, BR, 128), lambda i: (1, i, 0)),
            pl.BlockSpec((BR, 128), lambda i: (i, 0)),
            pl.BlockSpec((BR, 128), lambda i: (GI + i, 0)),
            pl.BlockSpec((BR, 1), lambda i: (i, 0)),
            pl.BlockSpec((1, D), lambda i: (0, 0)),
            pl.BlockSpec((D, 5), lambda i: (0, 0)),
            pl.BlockSpec((1, 5), lambda i: (0, 0)),
            pl.BlockSpec((5, D), lambda i: (0, 0)),
        ],
        out_specs=pl.BlockSpec((BR, D), lambda i: (i, 0)),
        out_shape=jax.ShapeDtypeStruct((N, D), jnp.float32),
    )(z, z, y, y, dinv, b, wa, ba, plist)


def _head_body(raw_ref, lab_ref, gate_ref, o_ref):
    raw = raw_ref[...]
    oh = (lax.broadcasted_iota(jnp.int32, (NB, B_IDX), 0)
          == lab_ref[...]).astype(jnp.float32)
    sums = _dot(oh, raw)
    cnts = jnp.sum(oh, axis=1, keepdims=True)
    ave = sums / jnp.maximum(cnts, 1.0) * gate_ref[0, 0]
    r = raw[0:1000, :]
    rn = jnp.sqrt(jnp.sum(r * r, axis=1, keepdims=True))
    an_sq = lax.dot_general(jnp.ones((1, D), jnp.float32), ave * ave,
                            (((1,), (1,)), ((), ())),
                            preferred_element_type=jnp.float32)
    dots = lax.dot_general(r, ave, (((1,), (1,)), ((), ())),
                           preferred_element_type=jnp.float32)
    denom = jnp.maximum(rn * jnp.sqrt(an_sq), 1e-8)
    ret = dots / denom
    m = jnp.max(ret, axis=1, keepdims=True)
    ex = jnp.exp(ret - m)
    o_ref[...] = ex / jnp.sum(ex, axis=1, keepdims=True)


def _head(raw, labels_pad, gate):
    return pl.pallas_call(
        _head_body,
        out_shape=jax.ShapeDtypeStruct((1000, NB), jnp.float32),
    )(raw, labels_pad, gate)


def kernel(x, params, edge_index, idx, labels, train):
    src = edge_index[0]
    dst = edge_index[1]

    pad = EP - E
    src_p = jnp.concatenate(
        [src, jnp.arange(pad, dtype=jnp.int32) % 128])
    dst_p = jnp.concatenate(
        [dst, N + (jnp.arange(pad, dtype=jnp.int32) % (ACC_ROWS - N))])
    src_r = src_p.reshape(NS, CONV_NB, CONV_B)
    src3 = jnp.stack([src_r, src_r + N])
    dst3 = dst_p.reshape(NS, CONV_NB, CONV_B)
    pk3 = src3 * (1 << PACK_SH) + dst3[None]

    pad_d = EP_D - E
    dst_dp = jnp.concatenate(
        [dst, N + (jnp.arange(pad_d, dtype=jnp.int32) % (DACC - N))])
    dstd = dst_dp.reshape(NC, NS, DEG_NB, CONV_B)

    degp = _sc_deg(dstd)
    deg = degp[0, :N] + degp[1, :N] + 1.0
    dinv = lax.rsqrt(deg).reshape(N, 1)

    Wg = params["gcn_W"]
    bg = [b.reshape(1, D) for b in params["gcn_b"]]
    w2 = params["gcn_weight2"].reshape(1, 1)

    ox = x
    y = _pre(x, Wg[0], dinv)
    for layer in params["cond"]:
        z = _sc_conv(y, pk3)
        e1, y = _postpre(z, y, dinv, bg[0], Wg[1], emit_e=True)
        z = _sc_conv(y, pk3)
        y = _postcond(z, y, dinv, bg[1], e1, w2, layer, ox, Wg[0])
    z = _sc_conv(y, pk3)
    (y,) = _postpre(z, y, dinv, bg[0], Wg[1], act="relu")
    z = _sc_conv(y, pk3)
    (y,) = _postpre(z, y, dinv, bg[1], Wg[2], act="relu")
    z = _sc_conv(y, pk3)
    embed = _posthead(z, y, dinv, bg[2], params["Wa"],
                      params["ba"].reshape(1, 5), params["p_list"])

    idx_p = jnp.concatenate(
        [idx, jnp.arange(B_IDX - 1000, dtype=jnp.int32)])
    raw = _sc_gather(embed, idx_p)
    labels_p = jnp.concatenate(
        [labels, jnp.full((B_IDX - 1000,), NB, jnp.int32)]).reshape(1, B_IDX)
    gate = jnp.where(train == 1, 1.0, 0.0).astype(jnp.float32).reshape(1, 1)
    return _head(raw, labels_p, gate)


# speedup vs baseline: 3.2703x; 1.0504x over previous
import functools

import jax
import jax.numpy as jnp
from jax import lax
from jax.experimental import pallas as pl
from jax.experimental.pallas import tpu as pltpu
from jax.experimental.pallas import tpu_sc as plsc

N = 10000
D = 256
NB = 10
E = 160000

NC, NS, L = 2, 16, 16

CONV_B = 128
CONV_NB = 79
EPT = CONV_NB * CONV_B
EP = NS * EPT
ACC_ROWS = 10112
ROWS_PT = ACC_ROWS // NS

DEG_NB = 40
EPT_D = DEG_NB * CONV_B
EP_D = NC * NS * EPT_D
DACC = 10240
DROWS_PT = DACC // NS

B_IDX = 1024

PACK_SH = 14
PACK_MASK = (1 << PACK_SH) - 1


_MESH = plsc.VectorSubcoreMesh(core_axis_name="c", subcore_axis_name="s",
                               num_cores=NC, num_subcores=NS)


@functools.partial(
    pl.kernel,
    out_type=jax.ShapeDtypeStruct((NC, DACC), jnp.float32),
    mesh=_MESH,
    scratch_types=[
        pltpu.VMEM_SHARED((DACC,), jnp.float32),
        pltpu.VMEM((DEG_NB, CONV_B), jnp.int32),
        pltpu.VMEM((DROWS_PT,), jnp.float32),
        pltpu.VMEM((CONV_B,), jnp.float32),
    ],
)
def _sc_deg(dst_hbm, degp_hbm, acc, dd, zb, ones):
    c = lax.axis_index("c")
    s = lax.axis_index("s")

    def zfill(i, _):
        zb[pl.ds(i * L, L)] = jnp.zeros((L,), jnp.float32)
        return 0

    lax.fori_loop(0, DROWS_PT // L, zfill, 0)
    for jj in range(CONV_B // L):
        ones[pl.ds(jj * L, L)] = jnp.ones((L,), jnp.float32)
    pltpu.sync_copy(zb, acc.at[pl.ds(s * DROWS_PT, DROWS_PT)])
    pltpu.sync_copy(dst_hbm.at[c, s], dd)
    plsc.subcore_barrier()

    def body(j, _):
        pltpu.sync_copy(ones, acc.at[dd.at[j]], add=True)
        return 0

    lax.fori_loop(0, DEG_NB, body, 0)
    plsc.subcore_barrier()
    pltpu.sync_copy(acc.at[pl.ds(s * DROWS_PT, DROWS_PT)],
                    degp_hbm.at[c, pl.ds(s * DROWS_PT, DROWS_PT)])


@functools.partial(
    pl.kernel,
    out_type=jax.ShapeDtypeStruct((NC, ACC_ROWS, 128), jnp.float32),
    mesh=_MESH,
    scratch_types=[
        pltpu.VMEM_SHARED((ACC_ROWS, 128), jnp.float32),
        pltpu.VMEM((CONV_NB, CONV_B), jnp.int32),
        pltpu.VMEM((2, CONV_B), jnp.int32),
        pltpu.VMEM((2, CONV_B), jnp.int32),
        pltpu.VMEM((CONV_B, 128), jnp.float32),
        pltpu.VMEM((CONV_B, 128), jnp.float32),
        pltpu.SemaphoreType.DMA,
        pltpu.SemaphoreType.DMA,
    ],
)
def _sc_conv(y_hbm, pk_hbm, z_hbm, acc, pb, sb, db, ra, rb, sema, semb):
    c = lax.axis_index("c")
    s = lax.axis_index("s")

    def zfill(i, _):
        for jj in range(128 // L):
            ra[i, pl.ds(jj * L, L)] = jnp.zeros((L,), jnp.float32)
        return 0

    lax.fori_loop(0, CONV_B, zfill, 0)
    base = s * ROWS_PT
    for k in range(4):
        pltpu.sync_copy(ra, acc.at[pl.ds(base + k * CONV_B, CONV_B)])
    pltpu.sync_copy(ra.at[pl.ds(0, ROWS_PT - 4 * CONV_B)],
                    acc.at[pl.ds(base + 4 * CONV_B, ROWS_PT - 4 * CONV_B)])
    pltpu.sync_copy(pk_hbm.at[c, s], pb)

    def unpack(j, t):
        for k in range(CONV_B // L):
            v = pb[j, pl.ds(k * L, L)]
            db[t, pl.ds(k * L, L)] = lax.bitwise_and(v, PACK_MASK)
            sb[t, pl.ds(k * L, L)] = lax.shift_right_logical(v, PACK_SH)

    plsc.subcore_barrier()

    unpack(0, 0)
    pltpu.async_copy(y_hbm.at[sb.at[0]], ra, sema)

    def body2(j2, _):
        j = j2 * 2
        unpack(j + 1, 1)
        pltpu.async_copy(y_hbm.at[sb.at[1]], rb, semb)
        pltpu.make_async_copy(y_hbm.at[sb.at[0]], ra, sema).wait()
        pltpu.sync_copy(ra, acc.at[db.at[0]], add=True)
        unpack(j + 2, 0)
        pltpu.async_copy(y_hbm.at[sb.at[0]], ra, sema)
        pltpu.make_async_copy(y_hbm.at[sb.at[1]], rb, semb).wait()
        pltpu.sync_copy(rb, acc.at[db.at[1]], add=True)
        return 0

    lax.fori_loop(0, (CONV_NB - 1) // 2, body2, 0)
    pltpu.make_async_copy(y_hbm.at[sb.at[0]], ra, sema).wait()
    pltpu.sync_copy(ra, acc.at[db.at[0]], add=True)

    plsc.subcore_barrier()
    for k in range(4):
        pltpu.sync_copy(acc.at[pl.ds(base + k * CONV_B, CONV_B)],
                        z_hbm.at[c, pl.ds(base + k * CONV_B, CONV_B)])
    pltpu.sync_copy(acc.at[pl.ds(base + 4 * CONV_B, ROWS_PT - 4 * CONV_B)],
                    z_hbm.at[c, pl.ds(base + 4 * CONV_B, ROWS_PT - 4 * CONV_B)])


@functools.partial(
    pl.kernel,
    out_type=jax.ShapeDtypeStruct((B_IDX, D), jnp.float32),
    mesh=_MESH,
    scratch_types=[
        pltpu.VMEM((B_IDX // (NC * NS),), jnp.int32),
        pltpu.VMEM((B_IDX // (NC * NS), D), jnp.float32),
        pltpu.SemaphoreType.DMA,
    ],
)
def _sc_gather(table_hbm, idx_hbm, out_hbm, idx_v, rows_v, sem):
    bpw = B_IDX // (NC * NS)
    wid = lax.axis_index("s") * NC + lax.axis_index("c")
    base = wid * bpw
    pltpu.sync_copy(idx_hbm.at[pl.ds(base, bpw)], idx_v)
    pltpu.async_copy(table_hbm.at[idx_v], rows_v, sem).wait()
    pltpu.sync_copy(rows_v, out_hbm.at[pl.ds(base, bpw)])


BR = 2000
GI = N // BR


def _dot(a, b):
    return jnp.dot(a, b, preferred_element_type=jnp.float32)


def _pre_body(x_ref, w_ref, dv_ref, o_ref):
    o_ref[...] = dv_ref[...] * _dot(x_ref[...], w_ref[...])


def _pre(x, w, dinv):
    return pl.pallas_call(
        _pre_body,
        grid=(GI, 2),
        in_specs=[
            pl.BlockSpec((BR, D), lambda i, j: (i, 0)),
            pl.BlockSpec((D, 128), lambda i, j: (0, j)),
            pl.BlockSpec((BR, 1), lambda i, j: (i, 0)),
        ],
        out_specs=pl.BlockSpec((BR, 128), lambda i, j: (j * GI + i, 0)),
        out_shape=jax.ShapeDtypeStruct((2 * N, 128), jnp.float32),
    )(x, w, dinv)


def _zy(z0_ref, z1_ref, y0_ref, y1_ref):
    zc = jnp.concatenate([z0_ref[0], z1_ref[0]], axis=1)
    yc = jnp.concatenate([y0_ref[...], y1_ref[...]], axis=1)
    return zc + yc


_ZY_SPECS = [
    pl.BlockSpec((1, BR, 128), lambda i, j: (0, i, 0)),
    pl.BlockSpec((1, BR, 128), lambda i, j: (1, i, 0)),
    pl.BlockSpec((BR, 128), lambda i, j: (i, 0)),
    pl.BlockSpec((BR, 128), lambda i, j: (GI + i, 0)),
]


def _postpre_body(act, emit_e, z0, z1, y0, y1, dv, b, wn, *outs):
    t = dv[...] * _zy(z0, z1, y0, y1) + b[...]
    if act == "relu":
        t = jnp.maximum(t, 0.0)
    if emit_e:
        outs[0][...] = t
    outs[-1][...] = dv[...] * _dot(t, wn[...])


def _postpre(z, y, dinv, b, wn, act=None, emit_e=False):
    out_shape = [jax.ShapeDtypeStruct((2 * N, 128), jnp.float32)]
    out_specs = [pl.BlockSpec((BR, 128), lambda i, j: (j * GI + i, 0))]
    if emit_e:
        out_shape.insert(0, jax.ShapeDtypeStruct((N, D), jnp.float32))
        out_specs.insert(0, pl.BlockSpec((BR, D), lambda i, j: (i, 0)))
    res = pl.pallas_call(
        functools.partial(_postpre_body, act, emit_e),
        grid=(GI, 2),
        in_specs=_ZY_SPECS + [
            pl.BlockSpec((BR, 1), lambda i, j: (i, 0)),
            pl.BlockSpec((1, D), lambda i, j: (0, 0)),
            pl.BlockSpec((D, 128), lambda i, j: (0, j)),
        ],
        out_specs=out_specs,
        out_shape=out_shape,
    )(z, z, y, y, dinv, b, wn)
    return res if emit_e else (res[0],)


def _postcond_body(z0, z1, y0, y1, dv, b, e1, w2, wi, bi, wh, bh, wo, bo,
                   ox, w0, yo_ref):
    e2 = dv[...] * _zy(z0, z1, y0, y1) + b[...] + e1[...]
    em = w2[0, 0] * e2
    h = _dot(em, wi[...]) + bi[...]
    h = jnp.where(h > 0, h, jnp.exp(jnp.minimum(h, 0.0)) - 1.0)
    h = _dot(h, wh[...]) + bh[...]
    h = jnp.where(h > 0, h, jnp.exp(jnp.minimum(h, 0.0)) - 1.0)
    prompt = _dot(h, wo[...]) + bo[...]
    xn = prompt * ox[...]
    yo_ref[...] = dv[...] * _dot(xn, w0[...])


def _postcond(z, y, dinv, b, e1, w2, layer, ox, w0):
    return pl.pallas_call(
        _postcond_body,
        grid=(GI, 2),
        in_specs=_ZY_SPECS + [
            pl.BlockSpec((BR, 1), lambda i, j: (i, 0)),
            pl.BlockSpec((1, D), lambda i, j: (0, 0)),
            pl.BlockSpec((BR, D), lambda i, j: (i, 0)),
            pl.BlockSpec((1, 1), lambda i, j: (0, 0)),
            pl.BlockSpec((D, D), lambda i, j: (0, 0)),
            pl.BlockSpec((1, D), lambda i, j: (0, 0)),
            pl.BlockSpec((D, D), lambda i, j: (0, 0)),
            pl.BlockSpec((1, D), lambda i, j: (0, 0)),
            pl.BlockSpec((D, D), lambda i, j: (0, 0)),
            pl.BlockSpec((1, D), lambda i, j: (0, 0)),
            pl.BlockSpec((BR, D), lambda i, j: (i, 0)),
            pl.BlockSpec((D, 128), lambda i, j: (0, j)),
        ],
        out_specs=pl.BlockSpec((BR, 128), lambda i, j: (j * GI + i, 0)),
        out_shape=jax.ShapeDtypeStruct((2 * N, 128), jnp.float32),
    )(z, z, y, y, dinv, b, e1, w2,
      layer["Wi"], layer["bi"].reshape(1, D),
      layer["Wh"], layer["bh"].reshape(1, D),
      layer["Wo"], layer["bo"].reshape(1, D), ox, w0)


def _posthead_body(z0, z1, y0, y1, dv, b, wa, ba, plist, o_ref):
    embed = dv[...] * _zy(z0, z1, y0, y1) + b[...]
    score = _dot(embed, wa[...]) + ba[...]
    m = jnp.max(score, axis=1, keepdims=True)
    ex = jnp.exp(score - m)
    weight = ex / jnp.sum(ex, axis=1, keepdims=True)
    o_ref[...] = embed + _dot(weight, plist[...])


def _posthead(z, y, dinv, b, wa, ba, plist):
    return pl.pallas_call(
        _posthead_body,
        grid=(GI,),
        in_specs=[
            pl.BlockSpec((1, BR, 128), lambda i: (0, i, 0)),
            pl.BlockSpec((1, BR, 128), lambda i: (1, i, 0)),
            pl.BlockSpec((BR, 128), lambda i: (i, 0)),
            pl.BlockSpec((BR, 128), lambda i: (GI + i, 0)),
            pl.BlockSpec((BR, 1), lambda i: (i, 0)),
            pl.BlockSpec((1, D), lambda i: (0, 0)),
            pl.BlockSpec((D, 5), lambda i: (0, 0)),
            pl.BlockSpec((1, 5), lambda i: (0, 0)),
            pl.BlockSpec((5, D), lambda i: (0, 0)),
        ],
        out_specs=pl.BlockSpec((BR, D), lambda i: (i, 0)),
        out_shape=jax.ShapeDtypeStruct((N, D), jnp.float32),
    )(z, z, y, y, dinv, b, wa, ba, plist)


def _head_body(raw_ref, lab_ref, gate_ref, o_ref):
    raw = raw_ref[...]
    oh = (lax.broadcasted_iota(jnp.int32, (NB, B_IDX), 0)
          == lab_ref[...]).astype(jnp.float32)
    sums = _dot(oh, raw)
    cnts = jnp.sum(oh, axis=1, keepdims=True)
    ave = sums / jnp.maximum(cnts, 1.0) * gate_ref[0, 0]
    r = raw[0:1000, :]
    rn = jnp.sqrt(jnp.sum(r * r, axis=1, keepdims=True))
    an_sq = lax.dot_general(jnp.ones((1, D), jnp.float32), ave * ave,
                            (((1,), (1,)), ((), ())),
                            preferred_element_type=jnp.float32)
    dots = lax.dot_general(r, ave, (((1,), (1,)), ((), ())),
                           preferred_element_type=jnp.float32)
    denom = jnp.maximum(rn * jnp.sqrt(an_sq), 1e-8)
    ret = dots / denom
    m = jnp.max(ret, axis=1, keepdims=True)
    ex = jnp.exp(ret - m)
    o_ref[...] = ex / jnp.sum(ex, axis=1, keepdims=True)


def _head(raw, labels_pad, gate):
    return pl.pallas_call(
        _head_body,
        out_shape=jax.ShapeDtypeStruct((1000, NB), jnp.float32),
    )(raw, labels_pad, gate)


def kernel(x, params, edge_index, idx, labels, train):
    src = edge_index[0]
    dst = edge_index[1]

    pad = EP - E
    src_p = jnp.concatenate(
        [src, jnp.arange(pad, dtype=jnp.int32) % 128])
    dst_p = jnp.concatenate(
        [dst, N + (jnp.arange(pad, dtype=jnp.int32) % (ACC_ROWS - N))])
    src_r = src_p.reshape(NS, CONV_NB, CONV_B)
    src3 = jnp.stack([src_r, src_r + N])
    dst3 = dst_p.reshape(NS, CONV_NB, CONV_B)
    pk3 = src3 * (1 << PACK_SH) + dst3[None]

    pad_d = EP_D - E
    dst_dp = jnp.concatenate(
        [dst, N + (jnp.arange(pad_d, dtype=jnp.int32) % (DACC - N))])
    dstd = dst_dp.reshape(NC, NS, DEG_NB, CONV_B)

    degp = _sc_deg(dstd)
    deg = degp[0, :N] + degp[1, :N] + 1.0
    dinv = lax.rsqrt(deg).reshape(N, 1)

    Wg = params["gcn_W"]
    bg = [b.reshape(1, D) for b in params["gcn_b"]]
    w2 = params["gcn_weight2"].reshape(1, 1)

    ox = x
    y = _pre(x, Wg[0], dinv)
    for layer in params["cond"]:
        z = _sc_conv(y, pk3)
        e1, y = _postpre(z, y, dinv, bg[0], Wg[1], emit_e=True)
        z = _sc_conv(y, pk3)
        y = _postcond(z, y, dinv, bg[1], e1, w2, layer, ox, Wg[0])
    z = _sc_conv(y, pk3)
    (y,) = _postpre(z, y, dinv, bg[0], Wg[1], act="relu")
    z = _sc_conv(y, pk3)
    (y,) = _postpre(z, y, dinv, bg[1], Wg[2], act="relu")
    z = _sc_conv(y, pk3)
    embed = _posthead(z, y, dinv, bg[2], params["Wa"],
                      params["ba"].reshape(1, 5), params["p_list"])

    idx_p = jnp.concatenate(
        [idx, jnp.arange(B_IDX - 1000, dtype=jnp.int32)])
    raw = _sc_gather(embed, idx_p)
    labels_p = jnp.concatenate(
        [labels, jnp.full((B_IDX - 1000,), NB, jnp.int32)]).reshape(1, B_IDX)
    gate = jnp.where(train == 1, 1.0, 0.0).astype(jnp.float32).reshape(1, 1)
    return _head(raw, labels_p, gate)


# speedup vs baseline: 3.3712x; 1.0309x over previous
import functools

import jax
import jax.numpy as jnp
from jax import lax
from jax.experimental import pallas as pl
from jax.experimental.pallas import tpu as pltpu
from jax.experimental.pallas import tpu_sc as plsc

N = 10000
D = 256
NB = 10
E = 160000

NC, NS, L = 2, 16, 16

CONV_B = 128
CONV_NB = 79
EPT = CONV_NB * CONV_B
EP = NS * EPT
ACC_ROWS = 10112
ROWS_PT = ACC_ROWS // NS

DEG_NB = 40
EPT_D = DEG_NB * CONV_B
EP_D = NC * NS * EPT_D
DACC = 10240
DROWS_PT = DACC // NS

B_IDX = 1024

PACK_SH = 14
PACK_MASK = (1 << PACK_SH) - 1


_MESH = plsc.VectorSubcoreMesh(core_axis_name="c", subcore_axis_name="s",
                               num_cores=NC, num_subcores=NS)


@functools.partial(
    pl.kernel,
    out_type=jax.ShapeDtypeStruct((NC, DACC), jnp.float32),
    mesh=_MESH,
    scratch_types=[
        pltpu.VMEM_SHARED((DACC,), jnp.float32),
        pltpu.VMEM((DEG_NB, CONV_B), jnp.int32),
        pltpu.VMEM((DROWS_PT,), jnp.float32),
        pltpu.VMEM((CONV_B,), jnp.float32),
    ],
)
def _sc_deg(dst_hbm, degp_hbm, acc, dd, zb, ones):
    c = lax.axis_index("c")
    s = lax.axis_index("s")

    def zfill(i, _):
        zb[pl.ds(i * L, L)] = jnp.zeros((L,), jnp.float32)
        return 0

    lax.fori_loop(0, DROWS_PT // L, zfill, 0)
    for jj in range(CONV_B // L):
        ones[pl.ds(jj * L, L)] = jnp.ones((L,), jnp.float32)
    pltpu.sync_copy(zb, acc.at[pl.ds(s * DROWS_PT, DROWS_PT)])
    pltpu.sync_copy(dst_hbm.at[c, s], dd)
    plsc.subcore_barrier()

    def body(j, _):
        pltpu.sync_copy(ones, acc.at[dd.at[j]], add=True)
        return 0

    lax.fori_loop(0, DEG_NB, body, 0)
    plsc.subcore_barrier()
    pltpu.sync_copy(acc.at[pl.ds(s * DROWS_PT, DROWS_PT)],
                    degp_hbm.at[c, pl.ds(s * DROWS_PT, DROWS_PT)])


@functools.partial(
    pl.kernel,
    out_type=jax.ShapeDtypeStruct((NC, ACC_ROWS, 128), jnp.float32),
    mesh=_MESH,
    scratch_types=[
        pltpu.VMEM_SHARED((ACC_ROWS, 128), jnp.float32),
        pltpu.VMEM((CONV_NB, CONV_B), jnp.int32),
        pltpu.VMEM((2, CONV_B), jnp.int32),
        pltpu.VMEM((2, CONV_B), jnp.int32),
        pltpu.VMEM((CONV_B, 128), jnp.float32),
        pltpu.VMEM((CONV_B, 128), jnp.float32),
        pltpu.SemaphoreType.DMA,
        pltpu.SemaphoreType.DMA,
    ],
)
def _sc_conv(y_hbm, pk_hbm, z_hbm, acc, pb, sb, db, ra, rb, sema, semb):
    c = lax.axis_index("c")
    s = lax.axis_index("s")

    def zfill(i, _):
        for jj in range(128 // L):
            ra[i, pl.ds(jj * L, L)] = jnp.zeros((L,), jnp.float32)
        return 0

    lax.fori_loop(0, CONV_B, zfill, 0)
    base = s * ROWS_PT
    for k in range(4):
        pltpu.sync_copy(ra, acc.at[pl.ds(base + k * CONV_B, CONV_B)])
    pltpu.sync_copy(ra.at[pl.ds(0, ROWS_PT - 4 * CONV_B)],
                    acc.at[pl.ds(base + 4 * CONV_B, ROWS_PT - 4 * CONV_B)])
    pltpu.sync_copy(pk_hbm.at[c, s], pb)

    def unpack(j, t):
        for k in range(CONV_B // L):
            v = pb[j, pl.ds(k * L, L)]
            db[t, pl.ds(k * L, L)] = lax.bitwise_and(v, PACK_MASK)
            sb[t, pl.ds(k * L, L)] = lax.shift_right_logical(v, PACK_SH)

    plsc.subcore_barrier()

    unpack(0, 0)
    pltpu.async_copy(y_hbm.at[sb.at[0]], ra, sema)

    def body2(j2, _):
        j = j2 * 2
        unpack(j + 1, 1)
        pltpu.async_copy(y_hbm.at[sb.at[1]], rb, semb)
        pltpu.make_async_copy(y_hbm.at[sb.at[0]], ra, sema).wait()
        pltpu.sync_copy(ra, acc.at[db.at[0]], add=True)
        unpack(j + 2, 0)
        pltpu.async_copy(y_hbm.at[sb.at[0]], ra, sema)
        pltpu.make_async_copy(y_hbm.at[sb.at[1]], rb, semb).wait()
        pltpu.sync_copy(rb, acc.at[db.at[1]], add=True)
        return 0

    lax.fori_loop(0, (CONV_NB - 1) // 2, body2, 0)
    pltpu.make_async_copy(y_hbm.at[sb.at[0]], ra, sema).wait()
    pltpu.sync_copy(ra, acc.at[db.at[0]], add=True)

    plsc.subcore_barrier()
    for k in range(4):
        pltpu.sync_copy(acc.at[pl.ds(base + k * CONV_B, CONV_B)],
                        z_hbm.at[c, pl.ds(base + k * CONV_B, CONV_B)])
    pltpu.sync_copy(acc.at[pl.ds(base + 4 * CONV_B, ROWS_PT - 4 * CONV_B)],
                    z_hbm.at[c, pl.ds(base + 4 * CONV_B, ROWS_PT - 4 * CONV_B)])


@functools.partial(
    pl.kernel,
    out_type=jax.ShapeDtypeStruct((B_IDX, D), jnp.float32),
    mesh=_MESH,
    scratch_types=[
        pltpu.VMEM((B_IDX // (NC * NS),), jnp.int32),
        pltpu.VMEM((B_IDX // (NC * NS), D), jnp.float32),
        pltpu.SemaphoreType.DMA,
    ],
)
def _sc_gather(table_hbm, idx_hbm, out_hbm, idx_v, rows_v, sem):
    bpw = B_IDX // (NC * NS)
    wid = lax.axis_index("s") * NC + lax.axis_index("c")
    base = wid * bpw
    pltpu.sync_copy(idx_hbm.at[pl.ds(base, bpw)], idx_v)
    pltpu.async_copy(table_hbm.at[idx_v], rows_v, sem).wait()
    pltpu.sync_copy(rows_v, out_hbm.at[pl.ds(base, bpw)])


BR = 5000
GI = N // BR


def _dot(a, b):
    return jnp.dot(a, b, preferred_element_type=jnp.float32)


def _pre_body(x_ref, w_ref, dv_ref, o_ref):
    o_ref[...] = dv_ref[...] * _dot(x_ref[...], w_ref[...])


def _pre(x, w, dinv):
    return pl.pallas_call(
        _pre_body,
        grid=(GI, 2),
        in_specs=[
            pl.BlockSpec((BR, D), lambda i, j: (i, 0)),
            pl.BlockSpec((D, 128), lambda i, j: (0, j)),
            pl.BlockSpec((BR, 1), lambda i, j: (i, 0)),
        ],
        out_specs=pl.BlockSpec((BR, 128), lambda i, j: (j * GI + i, 0)),
        out_shape=jax.ShapeDtypeStruct((2 * N, 128), jnp.float32),
    )(x, w, dinv)


def _zy(z0_ref, z1_ref, y0_ref, y1_ref):
    zc = jnp.concatenate([z0_ref[0], z1_ref[0]], axis=1)
    yc = jnp.concatenate([y0_ref[...], y1_ref[...]], axis=1)
    return zc + yc


_ZY_SPECS = [
    pl.BlockSpec((1, BR, 128), lambda i, j: (0, i, 0)),
    pl.BlockSpec((1, BR, 128), lambda i, j: (1, i, 0)),
    pl.BlockSpec((BR, 128), lambda i, j: (i, 0)),
    pl.BlockSpec((BR, 128), lambda i, j: (GI + i, 0)),
]


def _postpre_body(act, emit_e, z0, z1, y0, y1, dv, b, wn, *outs):
    t = dv[...] * _zy(z0, z1, y0, y1) + b[...]
    if act == "relu":
        t = jnp.maximum(t, 0.0)
    if emit_e:
        outs[0][...] = t
    outs[-1][...] = dv[...] * _dot(t, wn[...])


def _postpre(z, y, dinv, b, wn, act=None, emit_e=False):
    out_shape = [jax.ShapeDtypeStruct((2 * N, 128), jnp.float32)]
    out_specs = [pl.BlockSpec((BR, 128), lambda i, j: (j * GI + i, 0))]
    if emit_e:
        out_shape.insert(0, jax.ShapeDtypeStruct((N, D), jnp.float32))
        out_specs.insert(0, pl.BlockSpec((BR, D), lambda i, j: (i, 0)))
    res = pl.pallas_call(
        functools.partial(_postpre_body, act, emit_e),
        grid=(GI, 2),
        in_specs=_ZY_SPECS + [
            pl.BlockSpec((BR, 1), lambda i, j: (i, 0)),
            pl.BlockSpec((1, D), lambda i, j: (0, 0)),
            pl.BlockSpec((D, 128), lambda i, j: (0, j)),
        ],
        out_specs=out_specs,
        out_shape=out_shape,
    )(z, z, y, y, dinv, b, wn)
    return res if emit_e else (res[0],)


def _postcond_body(z0, z1, y0, y1, dv, b, e1, w2, wi, bi, wh, bh, wo, bo,
                   ox, w0, yo_ref):
    e2 = dv[...] * _zy(z0, z1, y0, y1) + b[...] + e1[...]
    em = w2[0, 0] * e2
    h = _dot(em, wi[...]) + bi[...]
    h = jnp.where(h > 0, h, jnp.exp(jnp.minimum(h, 0.0)) - 1.0)
    h = _dot(h, wh[...]) + bh[...]
    h = jnp.where(h > 0, h, jnp.exp(jnp.minimum(h, 0.0)) - 1.0)
    prompt = _dot(h, wo[...]) + bo[...]
    xn = prompt * ox[...]
    yo_ref[...] = dv[...] * _dot(xn, w0[...])


def _postcond(z, y, dinv, b, e1, w2, layer, ox, w0):
    return pl.pallas_call(
        _postcond_body,
        grid=(GI, 2),
        in_specs=_ZY_SPECS + [
            pl.BlockSpec((BR, 1), lambda i, j: (i, 0)),
            pl.BlockSpec((1, D), lambda i, j: (0, 0)),
            pl.BlockSpec((BR, D), lambda i, j: (i, 0)),
            pl.BlockSpec((1, 1), lambda i, j: (0, 0)),
            pl.BlockSpec((D, D), lambda i, j: (0, 0)),
            pl.BlockSpec((1, D), lambda i, j: (0, 0)),
            pl.BlockSpec((D, D), lambda i, j: (0, 0)),
            pl.BlockSpec((1, D), lambda i, j: (0, 0)),
            pl.BlockSpec((D, D), lambda i, j: (0, 0)),
            pl.BlockSpec((1, D), lambda i, j: (0, 0)),
            pl.BlockSpec((BR, D), lambda i, j: (i, 0)),
            pl.BlockSpec((D, 128), lambda i, j: (0, j)),
        ],
        out_specs=pl.BlockSpec((BR, 128), lambda i, j: (j * GI + i, 0)),
        out_shape=jax.ShapeDtypeStruct((2 * N, 128), jnp.float32),
    )(z, z, y, y, dinv, b, e1, w2,
      layer["Wi"], layer["bi"].reshape(1, D),
      layer["Wh"], layer["bh"].reshape(1, D),
      layer["Wo"], layer["bo"].reshape(1, D), ox, w0)


def _posthead_body(z0, z1, y0, y1, dv, b, wa, ba, plist, o_ref):
    embed = dv[...] * _zy(z0, z1, y0, y1) + b[...]
    score = _dot(embed, wa[...]) + ba[...]
    m = jnp.max(score, axis=1, keepdims=True)
    ex = jnp.exp(score - m)
    weight = ex / jnp.sum(ex, axis=1, keepdims=True)
    o_ref[...] = embed + _dot(weight, plist[...])


def _posthead(z, y, dinv, b, wa, ba, plist):
    return pl.pallas_call(
        _posthead_body,
        grid=(GI,),
        in_specs=[
            pl.BlockSpec((1, BR, 128), lambda i: (0, i, 0)),
            pl.BlockSpec((1, BR, 128), lambda i: (1, i, 0)),
            pl.BlockSpec((BR, 128), lambda i: (i, 0)),
            pl.BlockSpec((BR, 128), lambda i: (GI + i, 0)),
            pl.BlockSpec((BR, 1), lambda i: (i, 0)),
            pl.BlockSpec((1, D), lambda i: (0, 0)),
            pl.BlockSpec((D, 5), lambda i: (0, 0)),
            pl.BlockSpec((1, 5), lambda i: (0, 0)),
            pl.BlockSpec((5, D), lambda i: (0, 0)),
        ],
        out_specs=pl.BlockSpec((BR, D), lambda i: (i, 0)),
        out_shape=jax.ShapeDtypeStruct((N, D), jnp.float32),
    )(z, z, y, y, dinv, b, wa, ba, plist)


def _head_body(raw_ref, lab_ref, gate_ref, o_ref):
    raw = raw_ref[...]
    oh = (lax.broadcasted_iota(jnp.int32, (NB, B_IDX), 0)
          == lab_ref[...]).astype(jnp.float32)
    sums = _dot(oh, raw)
    cnts = jnp.sum(oh, axis=1, keepdims=True)
    ave = sums / jnp.maximum(cnts, 1.0) * gate_ref[0, 0]
    r = raw[0:1000, :]
    rn = jnp.sqrt(jnp.sum(r * r, axis=1, keepdims=True))
    an_sq = lax.dot_general(jnp.ones((1, D), jnp.float32), ave * ave,
                            (((1,), (1,)), ((), ())),
                            preferred_element_type=jnp.float32)
    dots = lax.dot_general(r, ave, (((1,), (1,)), ((), ())),
                           preferred_element_type=jnp.float32)
    denom = jnp.maximum(rn * jnp.sqrt(an_sq), 1e-8)
    ret = dots / denom
    m = jnp.max(ret, axis=1, keepdims=True)
    ex = jnp.exp(ret - m)
    o_ref[...] = ex / jnp.sum(ex, axis=1, keepdims=True)


def _head(raw, labels_pad, gate):
    return pl.pallas_call(
        _head_body,
        out_shape=jax.ShapeDtypeStruct((1000, NB), jnp.float32),
    )(raw, labels_pad, gate)


def kernel(x, params, edge_index, idx, labels, train):
    src = edge_index[0]
    dst = edge_index[1]

    pad = EP - E
    src_p = jnp.concatenate(
        [src, jnp.arange(pad, dtype=jnp.int32) % 128])
    dst_p = jnp.concatenate(
        [dst, N + (jnp.arange(pad, dtype=jnp.int32) % (ACC_ROWS - N))])
    src_r = src_p.reshape(NS, CONV_NB, CONV_B)
    src3 = jnp.stack([src_r, src_r + N])
    dst3 = dst_p.reshape(NS, CONV_NB, CONV_B)
    pk3 = src3 * (1 << PACK_SH) + dst3[None]

    pad_d = EP_D - E
    dst_dp = jnp.concatenate(
        [dst, N + (jnp.arange(pad_d, dtype=jnp.int32) % (DACC - N))])
    dstd = dst_dp.reshape(NC, NS, DEG_NB, CONV_B)

    degp = _sc_deg(dstd)
    deg = degp[0, :N] + degp[1, :N] + 1.0
    dinv = lax.rsqrt(deg).reshape(N, 1)

    Wg = params["gcn_W"]
    bg = [b.reshape(1, D) for b in params["gcn_b"]]
    w2 = params["gcn_weight2"].reshape(1, 1)

    ox = x
    y = _pre(x, Wg[0], dinv)
    for layer in params["cond"]:
        z = _sc_conv(y, pk3)
        e1, y = _postpre(z, y, dinv, bg[0], Wg[1], emit_e=True)
        z = _sc_conv(y, pk3)
        y = _postcond(z, y, dinv, bg[1], e1, w2, layer, ox, Wg[0])
    z = _sc_conv(y, pk3)
    (y,) = _postpre(z, y, dinv, bg[0], Wg[1], act="relu")
    z = _sc_conv(y, pk3)
    (y,) = _postpre(z, y, dinv, bg[1], Wg[2], act="relu")
    z = _sc_conv(y, pk3)
    embed = _posthead(z, y, dinv, bg[2], params["Wa"],
                      params["ba"].reshape(1, 5), params["p_list"])

    idx_p = jnp.concatenate(
        [idx, jnp.arange(B_IDX - 1000, dtype=jnp.int32)])
    raw = _sc_gather(embed, idx_p)
    labels_p = jnp.concatenate(
        [labels, jnp.full((B_IDX - 1000,), NB, jnp.int32)]).reshape(1, B_IDX)
    gate = jnp.where(train == 1, 1.0, 0.0).astype(jnp.float32).reshape(1, 1)
    return _head(raw, labels_p, gate)
